# baseline (speedup 1.0000x reference)
import jax
import jax.numpy as jnp
from jax import lax
from jax.experimental import pallas as pl
from jax.experimental.pallas import tpu as pltpu
from jax.experimental.pallas import tpu_sc as plsc

_NUM_CORES = 2
_NUM_SUBCORES = 16
_LANES = 16
_NUM_WORKERS = _NUM_CORES * _NUM_SUBCORES

_B = 16384
_D_T, _D_C, _D_R = 16, 8, 8
_D_OUT = _D_T + _D_C + _D_R
_N_COMB = 5 * 2 * 2
_B_PER_W = _B // _NUM_WORKERS
_GROUPS = _B_PER_W // _LANES
_CHUNK = 128


def _build_body(tw_ref, cw_ref, rw_ref, comb_ref):
    k = lax.broadcasted_iota(jnp.int32, (_N_COMB, 5), 0)
    v = lax.broadcasted_iota(jnp.int32, (_N_COMB, 5), 1)
    oh_t = (k // 4 == v).astype(jnp.float32)
    k2 = lax.broadcasted_iota(jnp.int32, (_N_COMB, 2), 0)
    v2 = lax.broadcasted_iota(jnp.int32, (_N_COMB, 2), 1)
    oh_c = ((k2 // 2) % 2 == v2).astype(jnp.float32)
    oh_r = (k2 % 2 == v2).astype(jnp.float32)
    t = jnp.dot(oh_t, tw_ref[...], preferred_element_type=jnp.float32)
    c = jnp.dot(oh_c, cw_ref[...], preferred_element_type=jnp.float32)
    r = jnp.dot(oh_r, rw_ref[...], preferred_element_type=jnp.float32)
    comb_ref[...] = jnp.concatenate(
        [t, c, r, jnp.zeros((_N_COMB, 128 - _D_OUT), jnp.float32)], axis=1)


def _lookup_body(tid_hbm, cid_hbm, rid_hbm, comb_hbm, out_hbm,
                 tid_v, cid_v, rid_v, idx_v, rows_v, sem):
    cid_ax = lax.axis_index("c")
    sid_ax = lax.axis_index("s")
    wid = sid_ax * _NUM_CORES + cid_ax
    base = wid * _B_PER_W

    pltpu.sync_copy(tid_hbm.at[pl.ds(base, _B_PER_W)], tid_v)
    pltpu.sync_copy(cid_hbm.at[pl.ds(base, _B_PER_W)], cid_v)
    pltpu.sync_copy(rid_hbm.at[pl.ds(base, _B_PER_W)], rid_v)

    for g in range(_GROUPS):
        chunk, off = divmod(g * _LANES, _CHUNK)
        sl = pl.ds(g * _LANES, _LANES)
        idx_v[chunk, pl.ds(off, _LANES)] = (
            tid_v[sl] * 4 + cid_v[sl] * 2 + rid_v[sl])

    for ch in range(_B_PER_W // _CHUNK):
        pltpu.async_copy(comb_hbm.at[idx_v.at[ch]],
                         rows_v.at[pl.ds(ch * _CHUNK, _CHUNK)], sem).wait()
    pltpu.sync_copy(rows_v, out_hbm.at[pl.ds(base, _B_PER_W)])


@jax.jit
def _run(tid, cid, rid, tw, cw, rw):
    comb = pl.pallas_call(
        _build_body,
        out_shape=jax.ShapeDtypeStruct((_N_COMB, 128), jnp.float32),
    )(tw, cw, rw)

    mesh = plsc.VectorSubcoreMesh(core_axis_name="c", subcore_axis_name="s")
    out128 = pl.kernel(
        _lookup_body,
        out_type=jax.ShapeDtypeStruct((_B, 128), jnp.float32),
        mesh=mesh,
        compiler_params=pltpu.CompilerParams(needs_layout_passes=False),
        scratch_types=[
            pltpu.VMEM((_B_PER_W,), jnp.int32),
            pltpu.VMEM((_B_PER_W,), jnp.int32),
            pltpu.VMEM((_B_PER_W,), jnp.int32),
            pltpu.VMEM((_B_PER_W // _CHUNK, _CHUNK), jnp.int32),
            pltpu.VMEM((_B_PER_W, 128), jnp.float32),
            pltpu.SemaphoreType.DMA,
        ],
    )(tid, cid, rid, comb)
    return out128[:, :_D_OUT]


def kernel(timepoint_ids, condition_ids, region_ids, timepoint_weight,
           condition_weight, region_weight):
    return _run(
        jnp.asarray(timepoint_ids, jnp.int32),
        jnp.asarray(condition_ids, jnp.int32),
        jnp.asarray(region_ids, jnp.int32),
        timepoint_weight,
        condition_weight,
        region_weight,
    )
